# baseline (speedup 1.0000x reference)
import functools

import jax
import jax.numpy as jnp
from jax import lax
from jax.experimental import pallas as pl
from jax.experimental.pallas import tpu as pltpu
from jax.experimental.pallas import tpu_sc as plsc

F32 = jnp.float32
I32 = jnp.int32

NC, NS, LANES = 2, 16, 16
NW = NC * NS
CHUNK = 192

N_ENT = 10000
N_USR = 20000
D = 128
DH = D // 2
N_REL = 15


def _sc_mesh():
  return plsc.VectorSubcoreMesh(core_axis_name="c", subcore_axis_name="s")


def _pack_bf16(x):
  u = (lax.bitcast_convert_type(x, jnp.uint32) + jnp.uint32(0x8000)) >> 16
  return lax.bitcast_convert_type((u[:, 1::2] << 16) | u[:, 0::2], jnp.float32)


def _unperm(x, nblk):
  n = x.shape[0]
  return (x.reshape(n, nblk, 2, 16).transpose(0, 1, 3, 2)
          .reshape(n, nblk * 32))


def _unpack2(pk):
  u = lax.bitcast_convert_type(pk, jnp.uint32)
  lo = lax.bitcast_convert_type(u << 16, F32)
  hi = lax.bitcast_convert_type(u & jnp.uint32(0xFFFF0000), F32)
  return lo, hi


def _edge_agg(head, tail, etype, gflat, m16, ent, rel16, zent):
  e_total = head.shape[0]
  ew = e_total // NW
  n_chunks = ew // CHUNK
  zblk = 1000

  @functools.partial(
      pl.kernel,
      out_type=jax.ShapeDtypeStruct((NC, N_ENT, D), F32),
      mesh=_sc_mesh(),
      compiler_params=pltpu.CompilerParams(use_tc_tiling_on_sc=False),
      scratch_types=[
          pltpu.VMEM((CHUNK,), I32),
          pltpu.VMEM((CHUNK,), I32),
          pltpu.VMEM((CHUNK,), I32),
          pltpu.VMEM((CHUNK,), I32),
          pltpu.VMEM((CHUNK,), I32),
          pltpu.VMEM((CHUNK,), I32),
          pltpu.VMEM((CHUNK,), F32),
          pltpu.VMEM((CHUNK,), F32),
          pltpu.VMEM((CHUNK,), F32),
          pltpu.VMEM((CHUNK, D // 2), F32),
          pltpu.VMEM((CHUNK, D), F32),
          pltpu.VMEM((16, D), F32),
          pltpu.VMEM((16,), F32),
          pltpu.SemaphoreType.DMA,
          pltpu.VMEM_SHARED((N_ENT, D), F32),
      ],
  )
  def k(head_h, tail_h, ty_h, g_h, m_h, ent_h, rel_h, z_h, out_h,
        h_v, t_v, ty_v, tt_v, ih_v, it_v, gh_s, gt_s, cf_v, rowp_v, rows_v,
        rel_v, m_v, sem_s, agg_sh):
    cid = lax.axis_index("c")
    sid = lax.axis_index("s")
    wid = sid * NC + cid

    @pl.when(sid < N_ENT // zblk)
    def _():
      pltpu.sync_copy(z_h.at[pl.ds(sid * zblk, zblk)],
                      agg_sh.at[pl.ds(sid * zblk, zblk)])
    pltpu.sync_copy(rel_h, rel_v)
    pltpu.sync_copy(m_h, m_v)
    plsc.subcore_barrier()

    mvec = m_v[...]

    def chunk_body(kk, _):
      @pl.when(kk > 0)
      def _():
        pltpu.make_async_copy(rows_v, agg_sh.at[h_v], sem_s).wait()

      base = wid * ew + kk * CHUNK
      pltpu.sync_copy(head_h.at[pl.ds(base, CHUNK)], h_v)
      pltpu.sync_copy(tail_h.at[pl.ds(base, CHUNK)], t_v)
      pltpu.sync_copy(ty_h.at[pl.ds(base, CHUNK)], ty_v)

      def idx_body(j, _):
        sl = pl.ds(j * LANES, LANES)
        et = ty_v[sl]
        tt = jnp.where(et == 0, N_REL - 1, et - 1)
        tt_v[sl] = tt
        ih_v[sl] = h_v[sl] * 16 + tt
        it_v[sl] = t_v[sl] * 16 + tt
        return 0

      lax.fori_loop(0, CHUNK // LANES, idx_body, 0, unroll=2)
      pltpu.sync_copy(ent_h.at[t_v], rowp_v)
      pltpu.sync_copy(g_h.at[ih_v], gh_s)
      pltpu.sync_copy(g_h.at[it_v], gt_s)

      def grp_body(j, _):
        sl = pl.ds(j * LANES, LANES)
        cf = jnp.exp(gh_s[sl] * gt_s[sl] - mvec)
        tt = tt_v[sl]
        for l in range(LANES):
          i = j * LANES + l
          w = cf[l]
          t = tt[l]
          for g in range(D // 32):
            lov, hiv = _unpack2(rowp_v[i, pl.ds(g * LANES, LANES)])
            lsl = pl.ds(g * 32, LANES)
            hsl = pl.ds(g * 32 + LANES, LANES)
            rows_v[i, lsl] = lov * rel_v[t, lsl] * w
            rows_v[i, hsl] = hiv * rel_v[t, hsl] * w
        return 0

      lax.fori_loop(0, CHUNK // LANES, grp_body, 0)
      pltpu.async_copy(rows_v, agg_sh.at[h_v], sem_s, add=True)
      return 0

    lax.fori_loop(0, n_chunks, chunk_body, 0)
    pltpu.make_async_copy(rows_v, agg_sh.at[h_v], sem_s).wait()
    plsc.subcore_barrier()

    @pl.when(sid < N_ENT // zblk)
    def _():
      pltpu.sync_copy(agg_sh.at[pl.ds(sid * zblk, zblk)],
                      out_h.at[cid, pl.ds(sid * zblk, zblk)])

  return k(head, tail, etype, gflat, m16, ent, rel16, zent)


def _user_agg(irows, icols, ivals, euf, zusr):
  nnzp = irows.shape[0]
  per_tile = nnzp // NS
  n_chunks = per_tile // CHUNK
  zblk = 2000

  @functools.partial(
      pl.kernel,
      out_type=jax.ShapeDtypeStruct((NC, N_USR, DH), F32),
      mesh=_sc_mesh(),
      compiler_params=pltpu.CompilerParams(use_tc_tiling_on_sc=False),
      scratch_types=[
          pltpu.VMEM((CHUNK,), I32),
          pltpu.VMEM((CHUNK,), I32),
          pltpu.VMEM((CHUNK,), I32),
          pltpu.VMEM((CHUNK,), F32),
          pltpu.VMEM((CHUNK, DH // 2), F32),
          pltpu.VMEM((CHUNK, DH), F32),
          pltpu.SemaphoreType.DMA,
          pltpu.VMEM_SHARED((N_USR, DH), F32),
      ],
  )
  def k(r_h, c_h, v_h, e_h, z_h, out_h, r_v, c_v, i_v, v_v, rowp_v, rows_v,
        sem_s, ua_sh):
    cid = lax.axis_index("c")
    sid = lax.axis_index("s")

    @pl.when(sid < N_USR // zblk)
    def _():
      pltpu.sync_copy(z_h.at[pl.ds(sid * zblk, zblk)],
                      ua_sh.at[pl.ds(sid * zblk, zblk)])
    plsc.subcore_barrier()

    def chunk_body(kk, _):
      @pl.when(kk > 0)
      def _():
        pltpu.make_async_copy(rows_v, ua_sh.at[r_v], sem_s).wait()

      base = sid * per_tile + kk * CHUNK
      pltpu.sync_copy(r_h.at[pl.ds(base, CHUNK)], r_v)
      pltpu.sync_copy(c_h.at[pl.ds(base, CHUNK)], c_v)
      pltpu.sync_copy(v_h.at[pl.ds(base, CHUNK)], v_v)

      def lane_body(j, _):
        sl = pl.ds(j * LANES, LANES)
        i_v[sl] = c_v[sl] * 2 + cid
        return 0

      lax.fori_loop(0, CHUNK // LANES, lane_body, 0, unroll=2)
      pltpu.sync_copy(e_h.at[i_v], rowp_v)

      def grp_body(j, _):
        vv = v_v[pl.ds(j * LANES, LANES)]
        for l in range(LANES):
          i = j * LANES + l
          v = vv[l]
          for g in range(DH // 32):
            lov, hiv = _unpack2(rowp_v[i, pl.ds(g * LANES, LANES)])
            rows_v[i, pl.ds(g * 32, LANES)] = lov * v
            rows_v[i, pl.ds(g * 32 + LANES, LANES)] = hiv * v
        return 0

      lax.fori_loop(0, CHUNK // LANES, grp_body, 0)
      pltpu.async_copy(rows_v, ua_sh.at[r_v], sem_s, add=True)
      return 0

    lax.fori_loop(0, n_chunks, chunk_body, 0)
    pltpu.make_async_copy(rows_v, ua_sh.at[r_v], sem_s).wait()
    plsc.subcore_barrier()

    @pl.when(sid < N_USR // zblk)
    def _():
      pltpu.sync_copy(ua_sh.at[pl.ds(sid * zblk, zblk)],
                      out_h.at[cid, pl.ds(sid * zblk, zblk)])

  return k(irows, icols, ivals, euf, zusr)


def _prep(ent, wsqt):
  def body(e_ref, w_ref, g_ref, m_ref):
    e = e_ref[...]
    g = jnp.dot(e * e, w_ref[...], preferred_element_type=F32)
    g_ref[...] = g
    m = jnp.max(g)
    m_ref[...] = jnp.full((8, 128), jnp.maximum(m * m - 30.0, 0.0), F32)

  return pl.pallas_call(
      body,
      out_shape=(jax.ShapeDtypeStruct((N_ENT, D), F32),
                 jax.ShapeDtypeStruct((8, 128), F32)),
  )(ent, wsqt)


def _ent_finalize(p0, p1, res):
  blk = 2000

  def body(a_ref, b_ref, r_ref, e_ref, o_ref):
    agg = a_ref[...] + b_ref[...]
    n = jnp.sqrt(jnp.sum(agg * agg, axis=1, keepdims=True))
    e = agg / jnp.maximum(n, 1e-12)
    e_ref[...] = e
    o_ref[...] = r_ref[...] + e

  grid = N_ENT // blk
  spec = pl.BlockSpec((blk, D), lambda i: (i, 0))
  return pl.pallas_call(
      body,
      grid=(grid,),
      in_specs=[spec, spec, spec],
      out_specs=(spec, spec),
      out_shape=(jax.ShapeDtypeStruct((N_ENT, D), F32),
                 jax.ShapeDtypeStruct((N_ENT, D), F32)),
  )(p0, p1, res)


def _user_finalize(usr, h0, h1, res, wt_pad, w_pad):
  blk = 2000

  def body(u_ref, h0_ref, h1_ref, r_ref, wt_ref, w_ref, un_ref, or_ref):
    u = u_ref[...]
    lg = jnp.dot(u, wt_ref[...], preferred_element_type=F32)
    col = lax.broadcasted_iota(I32, (blk, 128), 1)
    lg = jnp.where(col < N_REL, lg, -1e30)
    m = jnp.max(lg, axis=1, keepdims=True)
    ex = jnp.exp(lg - m)
    sm = ex / jnp.sum(ex, axis=1, keepdims=True)
    corr = jnp.dot(sm, w_ref[...], preferred_element_type=F32)
    ua = jnp.concatenate([h0_ref[...], h1_ref[...]], axis=1)
    ua = ua + corr * ua
    n = jnp.sqrt(jnp.sum(ua * ua, axis=1, keepdims=True))
    un = ua / jnp.maximum(n, 1e-12)
    un_ref[...] = un
    or_ref[...] = r_ref[...] + un

  grid = N_USR // blk
  spec = pl.BlockSpec((blk, D), lambda i: (i, 0))
  hspec = pl.BlockSpec((blk, DH), lambda i: (i, 0))
  wspec = pl.BlockSpec((128, 128), lambda i: (0, 0))
  return pl.pallas_call(
      body,
      grid=(grid,),
      in_specs=[spec, hspec, hspec, spec, wspec, wspec],
      out_specs=(spec, spec),
      out_shape=(jax.ShapeDtypeStruct((N_USR, D), F32),
                 jax.ShapeDtypeStruct((N_USR, D), F32)),
  )(usr, h0, h1, res, wt_pad, w_pad)


def kernel(user_emb, entity_emb, edge_index, edge_type, interact_rows,
           interact_cols, interact_vals, weight):
  head = edge_index[0].astype(I32)
  tail = edge_index[1].astype(I32)
  etype = edge_type.astype(I32)

  e_in = head.shape[0]
  ep = -(-e_in // (NW * CHUNK)) * (NW * CHUNK)
  epad = ep - e_in
  if epad:
    head = jnp.concatenate([head, (jnp.arange(epad, dtype=I32) * 13) % N_ENT])
    tail = jnp.concatenate([tail, (jnp.arange(epad, dtype=I32) * 17) % N_ENT])
    etype = jnp.concatenate([etype, jnp.full((epad,), 16, I32)])

  irows = interact_rows.astype(I32)
  icols = interact_cols.astype(I32)
  ivals = interact_vals.astype(F32)

  nnz = irows.shape[0]
  per_tile = -(-nnz // (NS * CHUNK)) * CHUNK
  nnzp = per_tile * NS
  padn = nnzp - nnz
  if padn:
    pr = (jnp.arange(padn, dtype=I32) * 37) % N_USR
    pc = (jnp.arange(padn, dtype=I32) * 29) % N_ENT
    irows = jnp.concatenate([irows, pr])
    icols = jnp.concatenate([icols, pc])
    ivals = jnp.concatenate([ivals, jnp.zeros((padn,), F32)])

  w = weight.astype(F32)
  wsq = w * w
  wsqt = jnp.zeros((D, 128), F32).at[:, :N_REL].set(wsq.T)
  wt_pad = jnp.zeros((D, 128), F32).at[:, :N_REL].set(w.T)
  w_pad = jnp.zeros((128, D), F32).at[:N_REL, :].set(w)
  blk = jnp.concatenate([jnp.arange(0, 32, 2), jnp.arange(1, 32, 2)])
  perm = (jnp.arange(D // 32)[:, None] * 32 + blk[None, :]).reshape(-1)
  rel16 = jnp.zeros((16, D), F32).at[:N_REL, :].set(w[:, perm])

  zent = jnp.zeros((N_ENT, D), F32)
  zusr = jnp.zeros((N_USR, DH), F32)

  ent = entity_emb.astype(F32)
  usr = user_emb.astype(F32)
  ent_res = ent
  usr_res = usr

  for _ in range(2):
    g, m8 = _prep(ent, wsqt)
    gflat = lax.slice(g, (0, 0), (N_ENT, 16)).reshape(N_ENT * 16)
    m16 = m8[0, :16]
    ent_pk = _pack_bf16(ent)
    euf_pk = ent_pk.reshape(2 * N_ENT, DH // 2)

    p_ent = _edge_agg(head, tail, etype, gflat, m16, ent_pk, rel16, zent)
    ua = _user_agg(irows, icols, ivals, euf_pk, zusr)

    ent, ent_res = _ent_finalize(_unperm(p_ent[0], 4), _unperm(p_ent[1], 4),
                                 ent_res)
    usr, usr_res = _user_finalize(usr, _unperm(ua[0], 2), _unperm(ua[1], 2),
                                  usr_res, wt_pad, w_pad)

  return ent_res, usr_res


# speedup vs baseline: 1.1812x; 1.1812x over previous
import functools

import jax
import jax.numpy as jnp
from jax import lax
from jax.experimental import pallas as pl
from jax.experimental.pallas import tpu as pltpu
from jax.experimental.pallas import tpu_sc as plsc

F32 = jnp.float32
I32 = jnp.int32

NC, NS, LANES = 2, 16, 16
NW = NC * NS
CE = 224
CU = 384

N_ENT = 10000
N_USR = 20000
D = 128
DH = D // 2
N_REL = 15


def _sc_mesh():
  return plsc.VectorSubcoreMesh(core_axis_name="c", subcore_axis_name="s")


def _pack_bf16(x):
  u = (lax.bitcast_convert_type(x, jnp.uint32) + jnp.uint32(0x8000)) >> 16
  return lax.bitcast_convert_type((u[:, 1::2] << 16) | u[:, 0::2], jnp.float32)


def _unperm(x, nblk):
  n = x.shape[0]
  return (x.reshape(n, nblk, 2, 16).transpose(0, 1, 3, 2)
          .reshape(n, nblk * 32))


def _unpack2(pk):
  u = lax.bitcast_convert_type(pk, jnp.uint32)
  lo = lax.bitcast_convert_type(u << 16, F32)
  hi = lax.bitcast_convert_type(u & jnp.uint32(0xFFFF0000), F32)
  return lo, hi


def _edge_agg(hty, gflat, m16, ent, rel16, zent):
  e_total = hty.shape[0]
  ew = e_total // NW
  n_chunks = ew // CE
  zblk = 1000

  @functools.partial(
      pl.kernel,
      out_type=jax.ShapeDtypeStruct((NC, N_ENT, D), F32),
      mesh=_sc_mesh(),
      compiler_params=pltpu.CompilerParams(use_tc_tiling_on_sc=False),
      scratch_types=[
          pltpu.VMEM((CE,), I32),
          pltpu.VMEM((CE,), I32),
          pltpu.VMEM((CE,), I32),
          pltpu.VMEM((CE,), I32),
          pltpu.VMEM((2 * CE,), I32),
          pltpu.VMEM((2 * CE,), F32),
          pltpu.VMEM((CE, D // 2), F32),
          pltpu.VMEM((CE, D), F32),
          pltpu.VMEM((16, D), F32),
          pltpu.VMEM((16,), F32),
          pltpu.SemaphoreType.DMA,
          pltpu.VMEM_SHARED((N_ENT, D), F32),
      ],
  )
  def k(hty_h, g_h, m_h, ent_h, rel_h, z_h, out_h,
        pk_v, h_v, t_v, tt_v, ix_v, gg_s, rowp_v, rows_v,
        rel_v, m_v, sem_s, agg_sh):
    cid = lax.axis_index("c")
    sid = lax.axis_index("s")
    wid = sid * NC + cid

    @pl.when(sid < N_ENT // zblk)
    def _():
      pltpu.sync_copy(z_h.at[pl.ds(sid * zblk, zblk)],
                      agg_sh.at[pl.ds(sid * zblk, zblk)])
    pltpu.sync_copy(rel_h, rel_v)
    pltpu.sync_copy(m_h, m_v)
    plsc.subcore_barrier()

    mvec = m_v[...]

    def chunk_body(kk, _):
      @pl.when(kk > 0)
      def _():
        pltpu.make_async_copy(rows_v, agg_sh.at[h_v], sem_s).wait()

      base = wid * ew + kk * CE
      pltpu.sync_copy(hty_h.at[pl.ds(base, CE)], pk_v)

      def idx_body(j, _):
        sl = pl.ds(j * LANES, LANES)
        u = lax.bitcast_convert_type(pk_v[sl], jnp.uint32)
        tt = (u >> 28).astype(I32)
        hh = ((u >> 14) & jnp.uint32(0x3FFF)).astype(I32)
        tl = (u & jnp.uint32(0x3FFF)).astype(I32)
        tt_v[sl] = tt
        h_v[sl] = hh
        t_v[sl] = tl
        ix_v[sl] = hh * 16 + tt
        ix_v[pl.ds(CE + j * LANES, LANES)] = tl * 16 + tt
        return 0

      lax.fori_loop(0, CE // LANES, idx_body, 0, unroll=2)
      pltpu.sync_copy(ent_h.at[t_v], rowp_v)
      pltpu.sync_copy(g_h.at[ix_v], gg_s)

      def grp_body(j, _):
        sl = pl.ds(j * LANES, LANES)
        cf = jnp.exp(gg_s[sl] * gg_s[pl.ds(CE + j * LANES, LANES)] - mvec)
        tt = tt_v[sl]
        for l in range(LANES):
          i = j * LANES + l
          w = cf[l]
          t = tt[l]
          for g in range(D // 32):
            lov, hiv = _unpack2(rowp_v[i, pl.ds(g * LANES, LANES)])
            lsl = pl.ds(g * 32, LANES)
            hsl = pl.ds(g * 32 + LANES, LANES)
            rows_v[i, lsl] = lov * rel_v[t, lsl] * w
            rows_v[i, hsl] = hiv * rel_v[t, hsl] * w
        return 0

      lax.fori_loop(0, CE // LANES, grp_body, 0)
      pltpu.async_copy(rows_v, agg_sh.at[h_v], sem_s, add=True)
      return 0

    lax.fori_loop(0, n_chunks, chunk_body, 0)
    pltpu.make_async_copy(rows_v, agg_sh.at[h_v], sem_s).wait()
    plsc.subcore_barrier()

    @pl.when(sid < N_ENT // zblk)
    def _():
      pltpu.sync_copy(agg_sh.at[pl.ds(sid * zblk, zblk)],
                      out_h.at[cid, pl.ds(sid * zblk, zblk)])

  return k(hty, gflat, m16, ent, rel16, zent)


def _user_agg(irc, ivals, euf, zusr):
  nnzp = irc.shape[0]
  per_tile = nnzp // NS
  n_chunks = per_tile // CU
  zblk = 2000

  @functools.partial(
      pl.kernel,
      out_type=jax.ShapeDtypeStruct((NC, N_USR, DH), F32),
      mesh=_sc_mesh(),
      compiler_params=pltpu.CompilerParams(use_tc_tiling_on_sc=False),
      scratch_types=[
          pltpu.VMEM((CU,), I32),
          pltpu.VMEM((CU,), I32),
          pltpu.VMEM((CU,), I32),
          pltpu.VMEM((CU,), F32),
          pltpu.VMEM((CU, DH // 2), F32),
          pltpu.VMEM((CU, DH), F32),
          pltpu.SemaphoreType.DMA,
          pltpu.VMEM_SHARED((N_USR, DH), F32),
      ],
  )
  def k(rc_h, v_h, e_h, z_h, out_h, pk_v, r_v, i_v, v_v, rowp_v, rows_v,
        sem_s, ua_sh):
    cid = lax.axis_index("c")
    sid = lax.axis_index("s")

    @pl.when(sid < N_USR // zblk)
    def _():
      pltpu.sync_copy(z_h.at[pl.ds(sid * zblk, zblk)],
                      ua_sh.at[pl.ds(sid * zblk, zblk)])
    plsc.subcore_barrier()

    def chunk_body(kk, _):
      @pl.when(kk > 0)
      def _():
        pltpu.make_async_copy(rows_v, ua_sh.at[r_v], sem_s).wait()

      base = sid * per_tile + kk * CU
      pltpu.sync_copy(rc_h.at[pl.ds(base, CU)], pk_v)
      pltpu.sync_copy(v_h.at[pl.ds(base, CU)], v_v)

      def lane_body(j, _):
        sl = pl.ds(j * LANES, LANES)
        u = pk_v[sl]
        r_v[sl] = u >> 14
        i_v[sl] = (u & 0x3FFF) * 2 + cid
        return 0

      lax.fori_loop(0, CU // LANES, lane_body, 0, unroll=2)
      pltpu.sync_copy(e_h.at[i_v], rowp_v)

      def grp_body(j, _):
        vv = v_v[pl.ds(j * LANES, LANES)]
        for l in range(LANES):
          i = j * LANES + l
          v = vv[l]
          for g in range(DH // 32):
            lov, hiv = _unpack2(rowp_v[i, pl.ds(g * LANES, LANES)])
            rows_v[i, pl.ds(g * 32, LANES)] = lov * v
            rows_v[i, pl.ds(g * 32 + LANES, LANES)] = hiv * v
        return 0

      lax.fori_loop(0, CU // LANES, grp_body, 0)
      pltpu.async_copy(rows_v, ua_sh.at[r_v], sem_s, add=True)
      return 0

    lax.fori_loop(0, n_chunks, chunk_body, 0)
    pltpu.make_async_copy(rows_v, ua_sh.at[r_v], sem_s).wait()
    plsc.subcore_barrier()

    @pl.when(sid < N_USR // zblk)
    def _():
      pltpu.sync_copy(ua_sh.at[pl.ds(sid * zblk, zblk)],
                      out_h.at[cid, pl.ds(sid * zblk, zblk)])

  return k(irc, ivals, euf, zusr)


def _prep(ent, wsqt):
  def body(e_ref, w_ref, g_ref, m_ref):
    e = e_ref[...]
    g = jnp.dot(e * e, w_ref[...], preferred_element_type=F32)
    g_ref[...] = g
    m = jnp.max(g)
    m_ref[...] = jnp.full((8, 128), jnp.maximum(m * m - 30.0, 0.0), F32)

  return pl.pallas_call(
      body,
      out_shape=(jax.ShapeDtypeStruct((N_ENT, D), F32),
                 jax.ShapeDtypeStruct((8, 128), F32)),
  )(ent, wsqt)


def _ent_finalize(p0, p1, res):
  blk = 2000

  def body(a_ref, b_ref, r_ref, e_ref, o_ref):
    agg = a_ref[...] + b_ref[...]
    n = jnp.sqrt(jnp.sum(agg * agg, axis=1, keepdims=True))
    e = agg / jnp.maximum(n, 1e-12)
    e_ref[...] = e
    o_ref[...] = r_ref[...] + e

  grid = N_ENT // blk
  spec = pl.BlockSpec((blk, D), lambda i: (i, 0))
  return pl.pallas_call(
      body,
      grid=(grid,),
      in_specs=[spec, spec, spec],
      out_specs=(spec, spec),
      out_shape=(jax.ShapeDtypeStruct((N_ENT, D), F32),
                 jax.ShapeDtypeStruct((N_ENT, D), F32)),
  )(p0, p1, res)


def _user_finalize(usr, h0, h1, res, wt_pad, w_pad):
  blk = 2000

  def body(u_ref, h0_ref, h1_ref, r_ref, wt_ref, w_ref, un_ref, or_ref):
    u = u_ref[...]
    lg = jnp.dot(u, wt_ref[...], preferred_element_type=F32)
    col = lax.broadcasted_iota(I32, (blk, 128), 1)
    lg = jnp.where(col < N_REL, lg, -1e30)
    m = jnp.max(lg, axis=1, keepdims=True)
    ex = jnp.exp(lg - m)
    sm = ex / jnp.sum(ex, axis=1, keepdims=True)
    corr = jnp.dot(sm, w_ref[...], preferred_element_type=F32)
    ua = jnp.concatenate([h0_ref[...], h1_ref[...]], axis=1)
    ua = ua + corr * ua
    n = jnp.sqrt(jnp.sum(ua * ua, axis=1, keepdims=True))
    un = ua / jnp.maximum(n, 1e-12)
    un_ref[...] = un
    or_ref[...] = r_ref[...] + un

  grid = N_USR // blk
  spec = pl.BlockSpec((blk, D), lambda i: (i, 0))
  hspec = pl.BlockSpec((blk, DH), lambda i: (i, 0))
  wspec = pl.BlockSpec((128, 128), lambda i: (0, 0))
  return pl.pallas_call(
      body,
      grid=(grid,),
      in_specs=[spec, hspec, hspec, spec, wspec, wspec],
      out_specs=(spec, spec),
      out_shape=(jax.ShapeDtypeStruct((N_USR, D), F32),
                 jax.ShapeDtypeStruct((N_USR, D), F32)),
  )(usr, h0, h1, res, wt_pad, w_pad)


def kernel(user_emb, entity_emb, edge_index, edge_type, interact_rows,
           interact_cols, interact_vals, weight):
  head = edge_index[0].astype(I32)
  tail = edge_index[1].astype(I32)
  etype = edge_type.astype(I32)
  tt = jnp.where(etype == 0, N_REL - 1, etype - 1)

  e_in = head.shape[0]
  ep = -(-e_in // (NW * CE)) * (NW * CE)
  epad = ep - e_in
  if epad:
    head = jnp.concatenate([head, (jnp.arange(epad, dtype=I32) * 13) % N_ENT])
    tail = jnp.concatenate([tail, (jnp.arange(epad, dtype=I32) * 17) % N_ENT])
    tt = jnp.concatenate([tt, jnp.full((epad,), 15, I32)])
  hty = (tt << 28) | (head << 14) | tail

  irows = interact_rows.astype(I32)
  icols = interact_cols.astype(I32)
  ivals = interact_vals.astype(F32)

  nnz = irows.shape[0]
  per_tile = -(-nnz // (NS * CU)) * CU
  nnzp = per_tile * NS
  padn = nnzp - nnz
  if padn:
    pr = (jnp.arange(padn, dtype=I32) * 37) % N_USR
    pc = (jnp.arange(padn, dtype=I32) * 29) % N_ENT
    irows = jnp.concatenate([irows, pr])
    icols = jnp.concatenate([icols, pc])
    ivals = jnp.concatenate([ivals, jnp.zeros((padn,), F32)])
  irc = (irows << 14) | icols

  w = weight.astype(F32)
  wsq = w * w
  wsqt = jnp.zeros((D, 128), F32).at[:, :N_REL].set(wsq.T)
  wt_pad = jnp.zeros((D, 128), F32).at[:, :N_REL].set(w.T)
  w_pad = jnp.zeros((128, D), F32).at[:N_REL, :].set(w)
  blk = jnp.concatenate([jnp.arange(0, 32, 2), jnp.arange(1, 32, 2)])
  perm = (jnp.arange(D // 32)[:, None] * 32 + blk[None, :]).reshape(-1)
  rel16 = jnp.zeros((16, D), F32).at[:N_REL, :].set(w[:, perm])

  zent = jnp.zeros((N_ENT, D), F32)
  zusr = jnp.zeros((N_USR, DH), F32)

  ent = entity_emb.astype(F32)
  usr = user_emb.astype(F32)
  ent_res = ent
  usr_res = usr

  for _ in range(2):
    g, m8 = _prep(ent, wsqt)
    gflat = lax.slice(g, (0, 0), (N_ENT, 16)).reshape(N_ENT * 16)
    m16 = m8[0, :16]
    ent_pk = _pack_bf16(ent)
    euf_pk = ent_pk.reshape(2 * N_ENT, DH // 2)

    p_ent = _edge_agg(hty, gflat, m16, ent_pk, rel16, zent)
    ua = _user_agg(irc, ivals, euf_pk, zusr)

    ent, ent_res = _ent_finalize(_unperm(p_ent[0], 4), _unperm(p_ent[1], 4),
                                 ent_res)
    usr, usr_res = _user_finalize(usr, _unperm(ua[0], 2), _unperm(ua[1], 2),
                                  usr_res, wt_pad, w_pad)

  return ent_res, usr_res


# speedup vs baseline: 1.2004x; 1.0162x over previous
import functools

import jax
import jax.numpy as jnp
from jax import lax
from jax.experimental import pallas as pl
from jax.experimental.pallas import tpu as pltpu
from jax.experimental.pallas import tpu_sc as plsc

F32 = jnp.float32
I32 = jnp.int32

NC, NS, LANES = 2, 16, 16
NW = NC * NS
CE = 224
CU = 448

N_ENT = 10000
N_USR = 20000
D = 128
DH = D // 2
N_REL = 15


def _sc_mesh():
  return plsc.VectorSubcoreMesh(core_axis_name="c", subcore_axis_name="s")


def _pack_bf16(x):
  u = (lax.bitcast_convert_type(x, jnp.uint32) + jnp.uint32(0x8000)) >> 16
  return lax.bitcast_convert_type((u[:, 1::2] << 16) | u[:, 0::2], jnp.float32)


def _unperm(x, nblk):
  n = x.shape[0]
  return (x.reshape(n, nblk, 2, 16).transpose(0, 1, 3, 2)
          .reshape(n, nblk * 32))


def _unpack2(pk):
  u = lax.bitcast_convert_type(pk, jnp.uint32)
  lo = lax.bitcast_convert_type(u << 16, F32)
  hi = lax.bitcast_convert_type(u & jnp.uint32(0xFFFF0000), F32)
  return lo, hi


def _edge_agg(hty, gflat, m16, ent, rel16, zent):
  e_total = hty.shape[0]
  ew = e_total // NW
  n_chunks = ew // CE
  zblk = 1000

  @functools.partial(
      pl.kernel,
      out_type=jax.ShapeDtypeStruct((NC, N_ENT, D), F32),
      mesh=_sc_mesh(),
      compiler_params=pltpu.CompilerParams(use_tc_tiling_on_sc=False),
      scratch_types=[
          pltpu.VMEM((CE,), I32),
          pltpu.VMEM((CE,), I32),
          pltpu.VMEM((CE,), I32),
          pltpu.VMEM((CE,), I32),
          pltpu.VMEM((2 * CE,), I32),
          pltpu.VMEM((2 * CE,), F32),
          pltpu.VMEM((CE, D // 2), F32),
          pltpu.VMEM((CE, D), F32),
          pltpu.VMEM((16, D), F32),
          pltpu.VMEM((16,), F32),
          pltpu.SemaphoreType.DMA,
          pltpu.VMEM_SHARED((N_ENT, D), F32),
      ],
  )
  def k(hty_h, g_h, m_h, ent_h, rel_h, z_h, out_h,
        pk_v, h_v, t_v, tt_v, ix_v, gg_s, rowp_v, rows_v,
        rel_v, m_v, sem_s, agg_sh):
    cid = lax.axis_index("c")
    sid = lax.axis_index("s")
    wid = sid * NC + cid

    @pl.when(sid < N_ENT // zblk)
    def _():
      pltpu.sync_copy(z_h.at[pl.ds(sid * zblk, zblk)],
                      agg_sh.at[pl.ds(sid * zblk, zblk)])
    pltpu.sync_copy(rel_h, rel_v)
    pltpu.sync_copy(m_h, m_v)
    plsc.subcore_barrier()

    mvec = m_v[...]

    def chunk_body(kk, _):
      @pl.when(kk > 0)
      def _():
        pltpu.make_async_copy(rows_v, agg_sh.at[h_v], sem_s).wait()

      base = wid * ew + kk * CE
      pltpu.sync_copy(hty_h.at[pl.ds(base, CE)], pk_v)

      def idx_body(j, _):
        sl = pl.ds(j * LANES, LANES)
        u = lax.bitcast_convert_type(pk_v[sl], jnp.uint32)
        tt = (u >> 28).astype(I32)
        hh = ((u >> 14) & jnp.uint32(0x3FFF)).astype(I32)
        tl = (u & jnp.uint32(0x3FFF)).astype(I32)
        tt_v[sl] = tt
        h_v[sl] = hh
        t_v[sl] = tl
        ix_v[sl] = hh * 16 + tt
        ix_v[pl.ds(CE + j * LANES, LANES)] = tl * 16 + tt
        return 0

      lax.fori_loop(0, CE // LANES, idx_body, 0, unroll=2)
      pltpu.sync_copy(ent_h.at[t_v], rowp_v)
      pltpu.sync_copy(g_h.at[ix_v], gg_s)

      def grp_body(j, _):
        sl = pl.ds(j * LANES, LANES)
        cf = jnp.exp(gg_s[sl] * gg_s[pl.ds(CE + j * LANES, LANES)] - mvec)
        tt = tt_v[sl]
        for l in range(LANES):
          i = j * LANES + l
          w = cf[l]
          t = tt[l]
          for g in range(D // 32):
            lov, hiv = _unpack2(rowp_v[i, pl.ds(g * LANES, LANES)])
            lsl = pl.ds(g * 32, LANES)
            hsl = pl.ds(g * 32 + LANES, LANES)
            rows_v[i, lsl] = lov * rel_v[t, lsl] * w
            rows_v[i, hsl] = hiv * rel_v[t, hsl] * w
        return 0

      lax.fori_loop(0, CE // LANES, grp_body, 0)
      pltpu.async_copy(rows_v, agg_sh.at[h_v], sem_s, add=True)
      return 0

    lax.fori_loop(0, n_chunks, chunk_body, 0)
    pltpu.make_async_copy(rows_v, agg_sh.at[h_v], sem_s).wait()
    plsc.subcore_barrier()

    @pl.when(sid < N_ENT // zblk)
    def _():
      pltpu.sync_copy(agg_sh.at[pl.ds(sid * zblk, zblk)],
                      out_h.at[cid, pl.ds(sid * zblk, zblk)])

  return k(hty, gflat, m16, ent, rel16, zent)


def _user_agg(irc, ivals, euf, zusr):
  nnzp = irc.shape[0]
  per_tile = nnzp // NS
  n_chunks = per_tile // CU
  zblk = 2000

  @functools.partial(
      pl.kernel,
      out_type=jax.ShapeDtypeStruct((NC, N_USR, DH), F32),
      mesh=_sc_mesh(),
      compiler_params=pltpu.CompilerParams(use_tc_tiling_on_sc=False),
      scratch_types=[
          pltpu.VMEM((CU,), I32),
          pltpu.VMEM((CU,), I32),
          pltpu.VMEM((CU,), I32),
          pltpu.VMEM((CU,), F32),
          pltpu.VMEM((CU, DH // 2), F32),
          pltpu.VMEM((CU, DH), F32),
          pltpu.SemaphoreType.DMA,
          pltpu.VMEM_SHARED((N_USR, DH), F32),
      ],
  )
  def k(rc_h, v_h, e_h, z_h, out_h, pk_v, r_v, i_v, v_v, rowp_v, rows_v,
        sem_s, ua_sh):
    cid = lax.axis_index("c")
    sid = lax.axis_index("s")

    @pl.when(sid < N_USR // zblk)
    def _():
      pltpu.sync_copy(z_h.at[pl.ds(sid * zblk, zblk)],
                      ua_sh.at[pl.ds(sid * zblk, zblk)])
    plsc.subcore_barrier()

    def chunk_body(kk, _):
      @pl.when(kk > 0)
      def _():
        pltpu.make_async_copy(rows_v, ua_sh.at[r_v], sem_s).wait()

      base = sid * per_tile + kk * CU
      pltpu.sync_copy(rc_h.at[pl.ds(base, CU)], pk_v)
      pltpu.sync_copy(v_h.at[pl.ds(base, CU)], v_v)

      def lane_body(j, _):
        sl = pl.ds(j * LANES, LANES)
        u = pk_v[sl]
        r_v[sl] = u >> 14
        i_v[sl] = (u & 0x3FFF) * 2 + cid
        return 0

      lax.fori_loop(0, CU // LANES, lane_body, 0, unroll=2)
      pltpu.sync_copy(e_h.at[i_v], rowp_v)

      def grp_body(j, _):
        vv = v_v[pl.ds(j * LANES, LANES)]
        for l in range(LANES):
          i = j * LANES + l
          v = vv[l]
          for g in range(DH // 32):
            lov, hiv = _unpack2(rowp_v[i, pl.ds(g * LANES, LANES)])
            rows_v[i, pl.ds(g * 32, LANES)] = lov * v
            rows_v[i, pl.ds(g * 32 + LANES, LANES)] = hiv * v
        return 0

      lax.fori_loop(0, CU // LANES, grp_body, 0)
      pltpu.async_copy(rows_v, ua_sh.at[r_v], sem_s, add=True)
      return 0

    lax.fori_loop(0, n_chunks, chunk_body, 0)
    pltpu.make_async_copy(rows_v, ua_sh.at[r_v], sem_s).wait()
    plsc.subcore_barrier()

    @pl.when(sid < N_USR // zblk)
    def _():
      pltpu.sync_copy(ua_sh.at[pl.ds(sid * zblk, zblk)],
                      out_h.at[cid, pl.ds(sid * zblk, zblk)])

  return k(irc, ivals, euf, zusr)


def _prep(ent, wsqt):
  def body(e_ref, w_ref, g_ref, m_ref):
    e = e_ref[...]
    g = jnp.dot(e * e, w_ref[...], preferred_element_type=F32)
    g_ref[...] = g
    m = jnp.max(g)
    m_ref[...] = jnp.full((8, 128), jnp.maximum(m * m - 30.0, 0.0), F32)

  return pl.pallas_call(
      body,
      out_shape=(jax.ShapeDtypeStruct((N_ENT, D), F32),
                 jax.ShapeDtypeStruct((8, 128), F32)),
  )(ent, wsqt)


def _ent_finalize(p0, p1, res):
  blk = 2000

  def body(a_ref, b_ref, r_ref, e_ref, o_ref):
    agg = a_ref[...] + b_ref[...]
    n = jnp.sqrt(jnp.sum(agg * agg, axis=1, keepdims=True))
    e = agg / jnp.maximum(n, 1e-12)
    e_ref[...] = e
    o_ref[...] = r_ref[...] + e

  grid = N_ENT // blk
  spec = pl.BlockSpec((blk, D), lambda i: (i, 0))
  return pl.pallas_call(
      body,
      grid=(grid,),
      in_specs=[spec, spec, spec],
      out_specs=(spec, spec),
      out_shape=(jax.ShapeDtypeStruct((N_ENT, D), F32),
                 jax.ShapeDtypeStruct((N_ENT, D), F32)),
  )(p0, p1, res)


def _user_finalize(usr, h0, h1, res, wt_pad, w_pad):
  blk = 2000

  def body(u_ref, h0_ref, h1_ref, r_ref, wt_ref, w_ref, un_ref, or_ref):
    u = u_ref[...]
    lg = jnp.dot(u, wt_ref[...], preferred_element_type=F32)
    col = lax.broadcasted_iota(I32, (blk, 128), 1)
    lg = jnp.where(col < N_REL, lg, -1e30)
    m = jnp.max(lg, axis=1, keepdims=True)
    ex = jnp.exp(lg - m)
    sm = ex / jnp.sum(ex, axis=1, keepdims=True)
    corr = jnp.dot(sm, w_ref[...], preferred_element_type=F32)
    ua = jnp.concatenate([h0_ref[...], h1_ref[...]], axis=1)
    ua = ua + corr * ua
    n = jnp.sqrt(jnp.sum(ua * ua, axis=1, keepdims=True))
    un = ua / jnp.maximum(n, 1e-12)
    un_ref[...] = un
    or_ref[...] = r_ref[...] + un

  grid = N_USR // blk
  spec = pl.BlockSpec((blk, D), lambda i: (i, 0))
  hspec = pl.BlockSpec((blk, DH), lambda i: (i, 0))
  wspec = pl.BlockSpec((128, 128), lambda i: (0, 0))
  return pl.pallas_call(
      body,
      grid=(grid,),
      in_specs=[spec, hspec, hspec, spec, wspec, wspec],
      out_specs=(spec, spec),
      out_shape=(jax.ShapeDtypeStruct((N_USR, D), F32),
                 jax.ShapeDtypeStruct((N_USR, D), F32)),
  )(usr, h0, h1, res, wt_pad, w_pad)


def kernel(user_emb, entity_emb, edge_index, edge_type, interact_rows,
           interact_cols, interact_vals, weight):
  head = edge_index[0].astype(I32)
  tail = edge_index[1].astype(I32)
  etype = edge_type.astype(I32)
  tt = jnp.where(etype == 0, N_REL - 1, etype - 1)

  e_in = head.shape[0]
  ep = -(-e_in // (NW * CE)) * (NW * CE)
  epad = ep - e_in
  if epad:
    head = jnp.concatenate([head, (jnp.arange(epad, dtype=I32) * 13) % N_ENT])
    tail = jnp.concatenate([tail, (jnp.arange(epad, dtype=I32) * 17) % N_ENT])
    tt = jnp.concatenate([tt, jnp.full((epad,), 15, I32)])
  hty = (tt << 28) | (head << 14) | tail

  irows = interact_rows.astype(I32)
  icols = interact_cols.astype(I32)
  ivals = interact_vals.astype(F32)

  nnz = irows.shape[0]
  per_tile = -(-nnz // (NS * CU)) * CU
  nnzp = per_tile * NS
  padn = nnzp - nnz
  if padn:
    pr = (jnp.arange(padn, dtype=I32) * 37) % N_USR
    pc = (jnp.arange(padn, dtype=I32) * 29) % N_ENT
    irows = jnp.concatenate([irows, pr])
    icols = jnp.concatenate([icols, pc])
    ivals = jnp.concatenate([ivals, jnp.zeros((padn,), F32)])
  irc = (irows << 14) | icols

  w = weight.astype(F32)
  wsq = w * w
  wsqt = jnp.zeros((D, 128), F32).at[:, :N_REL].set(wsq.T)
  wt_pad = jnp.zeros((D, 128), F32).at[:, :N_REL].set(w.T)
  w_pad = jnp.zeros((128, D), F32).at[:N_REL, :].set(w)
  blk = jnp.concatenate([jnp.arange(0, 32, 2), jnp.arange(1, 32, 2)])
  perm = (jnp.arange(D // 32)[:, None] * 32 + blk[None, :]).reshape(-1)
  rel16 = jnp.zeros((16, D), F32).at[:N_REL, :].set(w[:, perm])

  zent = jnp.zeros((N_ENT, D), F32)
  zusr = jnp.zeros((N_USR, DH), F32)

  ent = entity_emb.astype(F32)
  usr = user_emb.astype(F32)
  ent_res = ent
  usr_res = usr

  for _ in range(2):
    g, m8 = _prep(ent, wsqt)
    gflat = lax.slice(g, (0, 0), (N_ENT, 16)).reshape(N_ENT * 16)
    m16 = m8[0, :16]
    ent_pk = _pack_bf16(ent)
    euf_pk = ent_pk.reshape(2 * N_ENT, DH // 2)

    p_ent = _edge_agg(hty, gflat, m16, ent_pk, rel16, zent)
    ua = _user_agg(irc, ivals, euf_pk, zusr)

    ent, ent_res = _ent_finalize(_unperm(p_ent[0], 4), _unperm(p_ent[1], 4),
                                 ent_res)
    usr, usr_res = _user_finalize(usr, _unperm(ua[0], 2), _unperm(ua[1], 2),
                                  usr_res, wt_pad, w_pad)

  return ent_res, usr_res
